# baseline (speedup 1.0000x reference)
import functools

import jax
import jax.numpy as jnp
from jax import lax
from jax.experimental import pallas as pl
from jax.experimental.pallas import tpu as pltpu
from jax.experimental.pallas import tpu_sc as plsc

_VOCAB = 100000
_D = 64
_B = 16384
_L = 20

_NC = 2
_NS = 16
_NW = _NC * _NS
_BPW = _B // _NW
_G = 4
_CH = _G * _L
_NCHUNK = 2 * _BPW // _G
_NBUF = 4
_CCH = 128


def _sc_body(cw_hbm, idx_hbm, ctab_hbm, xtab_hbm,
             c_out, s_out,
             idxs_v, cw_v, crows, cout, idx_b, rows, out_v,
             ssem, csem, *sems):
    wid = lax.axis_index("s") * _NC + lax.axis_index("c")
    base = wid * _BPW

    idx_load = pltpu.async_copy(idx_hbm.at[pl.ds(base, _BPW)], idxs_v, ssem)
    cw_load = pltpu.async_copy(cw_hbm.at[pl.ds(base, _BPW)], cw_v, csem)
    cw_load.wait()

    def fire_center(h):
        for t in range(2):
            pltpu.async_copy(
                ctab_hbm.at[cw_v.at[pl.ds(h * 256 + t * _CCH, _CCH)]],
                crows.at[pl.ds(t * _CCH, _CCH)], csem)

    def drain_center():
        for t in range(2):
            pltpu.make_async_copy(
                ctab_hbm.at[cw_v.at[pl.ds(0, _CCH)]],
                crows.at[pl.ds(0, _CCH)], csem).wait()

    def repack_flush_center(h):
        def repack(r, carry):
            for k in range(2):
                for q in range(2):
                    cout[r, pl.ds(k * _D + q * 32, 32)] = (
                        crows[2 * r + k, pl.ds(q * 32, 32)])
            return carry

        lax.fori_loop(0, _CCH, repack, 0)
        pltpu.sync_copy(
            cout, c_out.at[pl.ds(wid * 256 + h * _CCH, _CCH)])

    fire_center(0)
    idx_load.wait()

    def compact(j, b):
        q = j & 127
        cb = (j >> 7) * _L
        for s in range(5):
            f = lax.iota(jnp.int32, 16) + (16 * s)
            g = lax.shift_right_logical(f * 3277, 16)
            row = g + q * _G
            col = (f - g * _L) + cb
            idx_b[b][pl.ds(16 * s, 16)] = plsc.load_gather(
                idxs_v, [row, col])

    def fire(j, b):
        compact(j, b)
        pltpu.async_copy(
            xtab_hbm.at[cw_v.at[pl.ds(0, _CH)]], rows[b], sems[b])

    def wait(b):
        pltpu.make_async_copy(
            xtab_hbm.at[cw_v.at[pl.ds(0, _CH)]], rows[b], sems[b]).wait()

    for b in range(_NBUF - 1):
        fire(b, b)

    drain_center()
    fire_center(1)
    repack_flush_center(0)
    drain_center()
    repack_flush_center(1)

    def process(j, b):
        for g in range(_G):
            gb = g * _L
            a0 = rows[b][gb, pl.ds(0, 32)]
            a1 = rows[b][gb, pl.ds(32, 32)]
            for r in range(1, _L):
                a0 = a0 + rows[b][gb + r, pl.ds(0, 32)]
                a1 = a1 + rows[b][gb + r, pl.ds(32, 32)]
            row = j * (_G // 2) + (g >> 1)
            col = (g & 1) * _D
            out_v[row, pl.ds(col, 32)] = a0
            out_v[row, pl.ds(col + 32, 32)] = a1

    def body(t, carry):
        for b in range(_NBUF):
            j = t * _NBUF + b
            jn = j + _NBUF - 1
            fb = (b + _NBUF - 1) % _NBUF

            @pl.when(jn < _NCHUNK)
            def _():
                fire(jn, fb)

            wait(b)
            process(j, b)
        return carry

    lax.fori_loop(0, _NCHUNK // _NBUF, body, 0)

    half = _BPW // 2
    pltpu.sync_copy(out_v.at[pl.ds(0, half)],
                    s_out.at[pl.ds(wid * half, half)])
    pltpu.sync_copy(out_v.at[pl.ds(half, half)],
                    s_out.at[pl.ds(_B // 2 + wid * half, half)])


_sc_gather = functools.partial(
    pl.kernel,
    mesh=plsc.VectorSubcoreMesh(core_axis_name="c", subcore_axis_name="s"),
    out_type=[
        jax.ShapeDtypeStruct((_B // 2, 2 * _D), jnp.bfloat16),
        jax.ShapeDtypeStruct((_B, 2 * _D), jnp.bfloat16),
    ],
    scratch_types=[
        pltpu.VMEM((_BPW, 128), jnp.int32),
        pltpu.VMEM((_BPW,), jnp.int32),
        pltpu.VMEM((256, _D), jnp.bfloat16),
        pltpu.VMEM((_CCH, 2 * _D), jnp.bfloat16),
        [pltpu.VMEM((_CH,), jnp.int32)] * _NBUF,
        [pltpu.VMEM((_CH, _D), jnp.bfloat16)] * _NBUF,
        pltpu.VMEM((_BPW, 2 * _D), jnp.bfloat16),
        pltpu.SemaphoreType.DMA,
        pltpu.SemaphoreType.DMA,
    ] + [pltpu.SemaphoreType.DMA] * _NBUF,
    compiler_params=pltpu.CompilerParams(use_tc_tiling_on_sc=False, needs_layout_passes=False),
)(_sc_body)


_BK = 4096
_GRID = _B // _BK


def _tc_body(c_ref, p_ref, n_ref, out_ref):
    i = pl.program_id(0)
    c = c_ref[...].astype(jnp.float32)
    p = p_ref[...].astype(jnp.float32)
    n = n_ref[...].astype(jnp.float32)

    def logsig(x):
        return jnp.minimum(x, 0.0) - jnp.log1p(jnp.exp(-jnp.abs(x)))

    def half_loss(sl):
        ch = c[:, sl]
        norm2 = jnp.sum(ch * ch, axis=1, keepdims=True)
        norm = jnp.sqrt(norm2)
        scale = jnp.where(norm > 1.0, 1.0 / (norm + 1e-7), 1.0)
        ps = jnp.sum(p[:, sl] * ch, axis=1, keepdims=True) * scale
        ns = jnp.sum(n[:, sl] * ch, axis=1, keepdims=True) * scale
        return -jnp.sum(logsig(ps) + logsig(-ns))

    part = half_loss(slice(0, _D)) + half_loss(slice(_D, 2 * _D))
    prev = jnp.where(i == 0, jnp.zeros((1, 1), jnp.float32), out_ref[...])
    total = prev + part
    out_ref[...] = jnp.where(i == _GRID - 1, total / _B, total)


_tc_epilogue = pl.pallas_call(
    _tc_body,
    grid=(_GRID,),
    in_specs=[
        pl.BlockSpec((_BK // 2, 2 * _D), lambda i: (i, 0)),
        pl.BlockSpec((_BK // 2, 2 * _D), lambda i: (i, 0)),
        pl.BlockSpec((_BK // 2, 2 * _D), lambda i: (i + _GRID, 0)),
    ],
    out_specs=pl.BlockSpec((1, 1), lambda i: (0, 0)),
    out_shape=jax.ShapeDtypeStruct((1, 1), jnp.float32),
)


def kernel(center_word, pos_context, neg_context, center_table, context_table):
    cw = center_word.astype(jnp.int32)
    z = jnp.zeros((_B, 128 - 2 * _L), jnp.int32)
    idxc = jnp.concatenate(
        [pos_context.astype(jnp.int32), neg_context.astype(jnp.int32), z],
        axis=1)
    ctab16 = center_table.astype(jnp.bfloat16)
    xtab16 = context_table.astype(jnp.bfloat16)
    c_rows, ctx_sum = _sc_gather(cw, idxc, ctab16, xtab16)
    out = _tc_epilogue(c_rows, ctx_sum, ctx_sum)
    return out[0, 0]


# speedup vs baseline: 1.0115x; 1.0115x over previous
import functools

import jax
import jax.numpy as jnp
from jax import lax
from jax.experimental import pallas as pl
from jax.experimental.pallas import tpu as pltpu
from jax.experimental.pallas import tpu_sc as plsc

_VOCAB = 100000
_D = 64
_B = 16384
_L = 20

_NC = 2
_NS = 16
_NW = _NC * _NS
_BPW = _B // _NW
_G = 4
_CH = _G * _L
_NCHUNK = 2 * _BPW // _G
_NBUF = 4
_CCH = 128


def _sc_body(cw_hbm, idx_hbm, ctab_hbm, xtab_hbm,
             c_out, s_out,
             idxs_v, cw_v, crows, cout, idx_b, rows, out_v,
             ssem, csem, *sems):
    wid = lax.axis_index("s") * _NC + lax.axis_index("c")
    base = wid * _BPW

    idx_load = pltpu.async_copy(idx_hbm.at[pl.ds(base, _BPW)], idxs_v, ssem)
    cw_load = pltpu.async_copy(cw_hbm.at[pl.ds(base, _BPW)], cw_v, csem)
    cw_load.wait()

    def fire_center(h):
        for t in range(2):
            pltpu.async_copy(
                ctab_hbm.at[cw_v.at[pl.ds(h * 256 + t * _CCH, _CCH)]],
                crows.at[pl.ds(t * _CCH, _CCH)], csem)

    def drain_center():
        for t in range(2):
            pltpu.make_async_copy(
                ctab_hbm.at[cw_v.at[pl.ds(0, _CCH)]],
                crows.at[pl.ds(0, _CCH)], csem).wait()

    def repack_flush_center(h):
        def repack(r, carry):
            for k in range(2):
                for q in range(2):
                    cout[r, pl.ds(k * _D + q * 32, 32)] = (
                        crows[2 * r + k, pl.ds(q * 32, 32)])
            return carry

        lax.fori_loop(0, _CCH, repack, 0)
        pltpu.sync_copy(
            cout, c_out.at[pl.ds(wid * 256 + h * _CCH, _CCH)])

    fire_center(0)
    idx_load.wait()

    def compact(j, b):
        q = j & 127
        cb = (j >> 7) * _L
        for s in range(5):
            f = lax.iota(jnp.int32, 16) + (16 * s)
            g = lax.shift_right_logical(f * 3277, 16)
            row = g + q * _G
            col = (f - g * _L) + cb
            idx_b[b][pl.ds(16 * s, 16)] = plsc.load_gather(
                idxs_v, [row, col])

    def fire(j, b):
        pltpu.async_copy(xtab_hbm.at[idx_b[b]], rows[b], sems[b])

    def wait(b):
        pltpu.make_async_copy(
            xtab_hbm.at[idx_b[b]], rows[b], sems[b]).wait()

    for b in range(_NBUF):
        compact(b, b)
    for b in range(_NBUF - 1):
        fire(b, b)

    drain_center()
    repack_flush_center(0)
    fire_center(1)
    drain_center()
    repack_flush_center(1)

    def process(j, b):
        for g in range(_G):
            gb = g * _L
            a0 = rows[b][gb, pl.ds(0, 32)]
            a1 = rows[b][gb, pl.ds(32, 32)]
            for r in range(1, _L):
                a0 = a0 + rows[b][gb + r, pl.ds(0, 32)]
                a1 = a1 + rows[b][gb + r, pl.ds(32, 32)]
            row = j * (_G // 2) + (g >> 1)
            col = (g & 1) * _D
            out_v[row, pl.ds(col, 32)] = a0
            out_v[row, pl.ds(col + 32, 32)] = a1

    def body(t, carry):
        for b in range(_NBUF):
            j = t * _NBUF + b
            jn = j + _NBUF - 1
            fb = (b + _NBUF - 1) % _NBUF

            @pl.when(jn < _NCHUNK)
            def _():
                fire(jn, fb)

            wait(b)

            @pl.when(jn + 1 < _NCHUNK)
            def _():
                compact(jn + 1, b)

            process(j, b)
        return carry

    lax.fori_loop(0, _NCHUNK // _NBUF, body, 0)

    half = _BPW // 2
    pltpu.sync_copy(out_v.at[pl.ds(0, half)],
                    s_out.at[pl.ds(wid * half, half)])
    pltpu.sync_copy(out_v.at[pl.ds(half, half)],
                    s_out.at[pl.ds(_B // 2 + wid * half, half)])


_sc_gather = functools.partial(
    pl.kernel,
    mesh=plsc.VectorSubcoreMesh(core_axis_name="c", subcore_axis_name="s"),
    out_type=[
        jax.ShapeDtypeStruct((_B // 2, 2 * _D), jnp.bfloat16),
        jax.ShapeDtypeStruct((_B, 2 * _D), jnp.bfloat16),
    ],
    scratch_types=[
        pltpu.VMEM((_BPW, 128), jnp.int32),
        pltpu.VMEM((_BPW,), jnp.int32),
        pltpu.VMEM((256, _D), jnp.bfloat16),
        pltpu.VMEM((_CCH, 2 * _D), jnp.bfloat16),
        [pltpu.VMEM((_CH,), jnp.int32)] * _NBUF,
        [pltpu.VMEM((_CH, _D), jnp.bfloat16)] * _NBUF,
        pltpu.VMEM((_BPW, 2 * _D), jnp.bfloat16),
        pltpu.SemaphoreType.DMA,
        pltpu.SemaphoreType.DMA,
    ] + [pltpu.SemaphoreType.DMA] * _NBUF,
    compiler_params=pltpu.CompilerParams(use_tc_tiling_on_sc=False, needs_layout_passes=False),
)(_sc_body)


_BK = 4096
_GRID = _B // _BK


def _tc_body(c_ref, p_ref, n_ref, out_ref):
    i = pl.program_id(0)
    c = c_ref[...].astype(jnp.float32)
    p = p_ref[...].astype(jnp.float32)
    n = n_ref[...].astype(jnp.float32)

    def logsig(x):
        return jnp.minimum(x, 0.0) - jnp.log1p(jnp.exp(-jnp.abs(x)))

    def half_loss(sl):
        ch = c[:, sl]
        norm2 = jnp.sum(ch * ch, axis=1, keepdims=True)
        norm = jnp.sqrt(norm2)
        scale = jnp.where(norm > 1.0, 1.0 / (norm + 1e-7), 1.0)
        ps = jnp.sum(p[:, sl] * ch, axis=1, keepdims=True) * scale
        ns = jnp.sum(n[:, sl] * ch, axis=1, keepdims=True) * scale
        return -jnp.sum(logsig(ps) + logsig(-ns))

    part = half_loss(slice(0, _D)) + half_loss(slice(_D, 2 * _D))
    prev = jnp.where(i == 0, jnp.zeros((1, 1), jnp.float32), out_ref[...])
    total = prev + part
    out_ref[...] = jnp.where(i == _GRID - 1, total / _B, total)


_tc_epilogue = pl.pallas_call(
    _tc_body,
    grid=(_GRID,),
    in_specs=[
        pl.BlockSpec((_BK // 2, 2 * _D), lambda i: (i, 0)),
        pl.BlockSpec((_BK // 2, 2 * _D), lambda i: (i, 0)),
        pl.BlockSpec((_BK // 2, 2 * _D), lambda i: (i + _GRID, 0)),
    ],
    out_specs=pl.BlockSpec((1, 1), lambda i: (0, 0)),
    out_shape=jax.ShapeDtypeStruct((1, 1), jnp.float32),
)


def kernel(center_word, pos_context, neg_context, center_table, context_table):
    cw = center_word.astype(jnp.int32)
    z = jnp.zeros((_B, 128 - 2 * _L), jnp.int32)
    idxc = jnp.concatenate(
        [pos_context.astype(jnp.int32), neg_context.astype(jnp.int32), z],
        axis=1)
    ctab16 = center_table.astype(jnp.bfloat16)
    xtab16 = context_table.astype(jnp.bfloat16)
    c_rows, ctx_sum = _sc_gather(cw, idxc, ctab16, xtab16)
    out = _tc_epilogue(c_rows, ctx_sum, ctx_sum)
    return out[0, 0]
